# baseline (speedup 1.0000x reference)
import jax
import jax.numpy as jnp
from jax import lax
from jax.experimental import pallas as pl
from jax.experimental.pallas import tpu as pltpu
from jax.experimental.pallas import tpu_sc as plsc

_N = 50000
_E = 800000
_B = 2048
_BM = 4096
_EPAD = 802816
_NP = 51200

_NC, _NS = 2, 16
_NW = _NC * _NS
_CHUNK = 64

_XOFF = (0, 4, 20, 36)
_XDIM = (4, 16, 16, 16)


def _sc_gather2(table, idx_r, idx_c, nbuf):
    M = idx_r.shape[0]
    per_w = M // _NW
    n_chunks = per_w // _CHUNK
    n_outer = n_chunks // nbuf
    assert per_w * _NW == M and n_outer * nbuf == n_chunks

    mesh = plsc.VectorSubcoreMesh(core_axis_name="c", subcore_axis_name="s")

    def body(table_hbm, ir_hbm, ic_hbm, gr_hbm, gc_hbm,
             ir_v, ic_v, rr_v, rc_v, sem_i, sem_g, sem_o):
        wid = lax.axis_index("s") * _NC + lax.axis_index("c")
        wbase = wid * per_w

        def outer(g, carry):
            base = wbase + g * (nbuf * _CHUNK)
            for b in range(nbuf):
                sl = pl.ds(base + b * _CHUNK, _CHUNK)
                pltpu.async_copy(ir_hbm.at[sl], ir_v.at[b], sem_i)
                pltpu.async_copy(ic_hbm.at[sl], ic_v.at[b], sem_i)
            for b in range(nbuf):
                pltpu.make_async_copy(
                    ir_hbm.at[pl.ds(base + b * _CHUNK, _CHUNK)], ir_v.at[b],
                    sem_i).wait()
                pltpu.make_async_copy(
                    ic_hbm.at[pl.ds(base + b * _CHUNK, _CHUNK)], ic_v.at[b],
                    sem_i).wait()
                pltpu.async_copy(table_hbm.at[ir_v.at[b]], rr_v.at[b], sem_g)
                pltpu.async_copy(table_hbm.at[ic_v.at[b]], rc_v.at[b], sem_g)
            for b in range(nbuf):
                sl = pl.ds(base + b * _CHUNK, _CHUNK)
                pltpu.make_async_copy(
                    table_hbm.at[ir_v.at[b]], rr_v.at[b], sem_g).wait()
                pltpu.make_async_copy(
                    table_hbm.at[ic_v.at[b]], rc_v.at[b], sem_g).wait()
                pltpu.async_copy(rr_v.at[b], gr_hbm.at[sl], sem_o)
                pltpu.async_copy(rc_v.at[b], gc_hbm.at[sl], sem_o)
            for b in range(nbuf):
                sl = pl.ds(base + b * _CHUNK, _CHUNK)
                pltpu.make_async_copy(rr_v.at[b], gr_hbm.at[sl], sem_o).wait()
                pltpu.make_async_copy(rc_v.at[b], gc_hbm.at[sl], sem_o).wait()
            return carry

        lax.fori_loop(0, n_outer, outer, 0)

    return pl.kernel(
        body,
        mesh=mesh,
        out_type=(jax.ShapeDtypeStruct((M, 128), jnp.float32),
                  jax.ShapeDtypeStruct((M, 128), jnp.float32)),
        scratch_types=[
            pltpu.VMEM((nbuf, _CHUNK), jnp.int32),
            pltpu.VMEM((nbuf, _CHUNK), jnp.int32),
            pltpu.VMEM((nbuf, _CHUNK, 128), jnp.float32),
            pltpu.VMEM((nbuf, _CHUNK, 128), jnp.float32),
            pltpu.SemaphoreType.DMA,
            pltpu.SemaphoreType.DMA,
            pltpu.SemaphoreType.DMA,
        ],
    )(table, idx_r, idx_c)


def _relu(v):
    return jnp.maximum(v, 0.0)


def _dot(x, w):
    return lax.dot_general(x, w, (((1,), (0,)), ((), ())),
                           preferred_element_type=jnp.float32)


def _log_softmax(t):
    m = jnp.max(t, axis=1, keepdims=True)
    return t - (m + jnp.log(jnp.sum(jnp.exp(t - m), axis=1, keepdims=True)))


_LKEYS = ("w128", "wc", "be1", "we2", "be2", "wne", "bn1", "wn2", "bn2")
_CKEYS = ("cw1", "cb1", "cw2", "cb2", "ew1", "eb1", "ew2", "eb2")


def _meta_layer(Gcat, ea, x, W, eh):
    m1 = _dot(Gcat, W["w128"])
    he = _relu(m1[:, :eh] + _dot(ea, W["wc"]) + W["be1"])
    ea_new = _dot(he, W["we2"]) + W["be2"]
    if x is not None:
        ea_new = ea_new + ea
    hn = _relu(m1[:, eh:] + _dot(ea_new, W["wne"]) + W["bn1"])
    x_new = _dot(hn, W["wn2"]) + W["bn2"]
    if x is not None:
        x_new = x_new + x
    return ea_new, x_new


def _wrefs(keys, refs):
    return {k: r[...] for k, r in zip(keys, refs)}


def _full_specs(arrs):
    specs = []
    for a in arrs:
        nd = a.ndim
        specs.append(pl.BlockSpec(a.shape, lambda i, _nd=nd: (0,) * _nd))
    return specs


def _prefix_layer(Gr, Gc, ea, xprev, w, eh):
    wlist = [w[k] for k in _LKEYS]
    residual = xprev is not None

    def body(gr_ref, gc_ref, ea_ref, *rest):
        if residual:
            xp_ref, rest = rest[0], rest[1:]
        wr = rest[:len(_LKEYS)]
        ea_o, x_o = rest[len(_LKEYS)], rest[len(_LKEYS) + 1]
        W = _wrefs(_LKEYS, wr)
        x = xp_ref[...] if residual else None
        gcat = jnp.concatenate([gr_ref[...][:, :64], gc_ref[...][:, :64]],
                               axis=1)
        ea_new, x_new = _meta_layer(gcat, ea_ref[...], x, W, eh)
        ea_o[...] = ea_new
        x_o[...] = x_new

    ins = [Gr, Gc, ea] + ([xprev] if residual else []) + wlist
    in_specs = [
        pl.BlockSpec((_B, 128), lambda i: (i, 0)),
        pl.BlockSpec((_B, 128), lambda i: (i, 0)),
        pl.BlockSpec((_B, ea.shape[1]), lambda i: (i, 0)),
    ]
    if residual:
        in_specs.append(pl.BlockSpec((_B, 16), lambda i: (i, 0)))
    in_specs += _full_specs(wlist)
    return pl.pallas_call(
        body,
        grid=(_NP // _B,),
        in_specs=in_specs,
        out_specs=(pl.BlockSpec((_B, 16), lambda i: (i, 0)),
                   pl.BlockSpec((_B, 16), lambda i: (i, 0))),
        out_shape=(jax.ShapeDtypeStruct((_NP, 16), jnp.float32),
                   jax.ShapeDtypeStruct((_NP, 16), jnp.float32)),
    )(*ins)


def _mega(Gr, Gc, ea0, layers, cls, blk0, nblk, ne):
    wlist = [l[k] for l in layers for k in _LKEYS] + [cls[k] for k in _CKEYS]
    nl = len(_LKEYS)

    def body(gr_ref, gc_ref, ea_ref, *rest):
        Ws = [_wrefs(_LKEYS, rest[i * nl:(i + 1) * nl]) for i in range(4)]
        C = _wrefs(_CKEYS, rest[4 * nl:4 * nl + len(_CKEYS)])
        no_ref, eo_ref = rest[-2], rest[-1]
        gcat = jnp.concatenate([gr_ref[...][:, :64], gc_ref[...][:, :64]],
                               axis=1)
        ea, x = ea_ref[...], None
        for i in range(4):
            ea, x = _meta_layer(gcat, ea, x, Ws[i], 32 if i == 0 else 64)
        hn = _relu(_dot(x, C["cw1"]) + C["cb1"])
        tn = _relu(_dot(hn, C["cw2"]) + C["cb2"])
        he = _relu(_dot(ea, C["ew1"]) + C["eb1"])
        te = _relu(_dot(he, C["ew2"]) + C["eb2"])
        no_ref[...] = _log_softmax(tn)
        eo_ref[...] = _log_softmax(te)

    in_specs = [
        pl.BlockSpec((_BM, 128), lambda i: (i, 0)),
        pl.BlockSpec((_BM, 128), lambda i: (i, 0)),
        pl.BlockSpec((_BM, 6), lambda i: (i + blk0, 0)),
    ] + _full_specs(wlist)
    return pl.pallas_call(
        body,
        grid=(nblk,),
        in_specs=in_specs,
        out_specs=(pl.BlockSpec((_BM, 2), lambda i: (i, 0)),
                   pl.BlockSpec((_BM, 4), lambda i: (i, 0))),
        out_shape=(jax.ShapeDtypeStruct((ne, 2), jnp.float32),
                   jax.ShapeDtypeStruct((ne, 4), jnp.float32)),
    )(Gr, Gc, ea0, *wlist)


def _prep_weights(p):
    layers = []
    for i in (1, 2, 3, 4):
        eT = p["e%dW1" % i].T
        nT = p["n%dW1" % i].T
        d = _XDIM[i - 1]
        lo = _XOFF[i - 1]
        eh = eT.shape[1]
        nh = nT.shape[1]
        w128 = jnp.zeros((128, eh + nh), jnp.float32)
        w128 = w128.at[lo:lo + d, :eh].set(eT[0:d])
        w128 = w128.at[64 + lo:64 + lo + d, :eh].set(eT[d:2 * d])
        w128 = w128.at[64 + lo:64 + lo + d, eh:].set(nT[0:d])
        layers.append(dict(
            w128=w128, wc=eT[2 * d:], be1=p["e%db1" % i][None],
            we2=p["e%dW2" % i].T, be2=p["e%db2" % i][None],
            wne=nT[d:], bn1=p["n%db1" % i][None],
            wn2=p["n%dW2" % i].T, bn2=p["n%db2" % i][None],
        ))
    cls = dict(
        cw1=p["cnW1"].T, cb1=p["cnb1"][None],
        cw2=p["cnW2"].T, cb2=p["cnb2"][None],
        ew1=p["ceW1"].T, eb1=p["ceb1"][None],
        ew2=p["ceW2"].T, eb2=p["ceb2"][None],
    )
    return layers, cls


def kernel(x, edge_attr, edge_index, params):
    layers, cls = _prep_weights(params)
    row = edge_index[0]
    col = edge_index[1]

    row_f = jnp.pad(row, (0, _EPAD - _E))
    col_f = jnp.pad(col, (0, _EPAD - _E))
    row_p = jnp.pad(row[:_N], (0, _NP - _N))
    col_p = jnp.pad(col[:_N], (0, _NP - _N))

    T = jnp.zeros((_N, 128), jnp.float32).at[:, 0:4].set(x)
    ea_p = jnp.pad(edge_attr[:_N], ((0, _NP - _N), (0, 0)))
    xp = None
    for i in range(3):
        Grp, Gcp = _sc_gather2(T, row_p, col_p, nbuf=5)
        ea_p, xp = _prefix_layer(Grp, Gcp, ea_p, xp, layers[i],
                                 32 if i == 0 else 64)
        o = _XOFF[i + 1]
        T = T.at[:, o:o + 16].set(xp[:_N])

    q = _EPAD // 4
    nodes, edges = [], []
    for k in range(4):
        lo, hi = k * q, (k + 1) * q
        Grk, Gck = _sc_gather2(T, row_f[lo:hi], col_f[lo:hi], nbuf=7)
        ne = min(hi, _E) - lo
        nk, ek = _mega(Grk, Gck, edge_attr, layers, cls, k * 49, 49, ne)
        nodes.append(nk)
        edges.append(ek)
    return (jnp.concatenate(nodes), jnp.concatenate(edges))


# speedup vs baseline: 1.0051x; 1.0051x over previous
import jax
import jax.numpy as jnp
from jax import lax
from jax.experimental import pallas as pl
from jax.experimental.pallas import tpu as pltpu
from jax.experimental.pallas import tpu_sc as plsc

_N = 50000
_E = 800000
_B = 2048
_BM = 4096
_EPAD = 802816
_NP = 51200

_NC, _NS = 2, 16
_NW = _NC * _NS
_CHUNK = 64

_XOFF = (0, 4, 20, 36)
_XDIM = (4, 16, 16, 16)


def _sc_gather2(table, idx_r, idx_c, nbuf):
    M = idx_r.shape[0]
    per_w = M // _NW
    n_chunks = per_w // _CHUNK
    n_outer = n_chunks // nbuf
    assert per_w * _NW == M and n_outer * nbuf == n_chunks

    mesh = plsc.VectorSubcoreMesh(core_axis_name="c", subcore_axis_name="s")

    def body(table_hbm, ir_hbm, ic_hbm, gr_hbm, gc_hbm,
             ir_v, ic_v, rr_v, rc_v, sem_i, sem_g, sem_o):
        wid = lax.axis_index("s") * _NC + lax.axis_index("c")
        wbase = wid * per_w

        def outer(g, carry):
            base = wbase + g * (nbuf * _CHUNK)
            for b in range(nbuf):
                sl = pl.ds(base + b * _CHUNK, _CHUNK)
                pltpu.async_copy(ir_hbm.at[sl], ir_v.at[b], sem_i)
                pltpu.async_copy(ic_hbm.at[sl], ic_v.at[b], sem_i)
            for b in range(nbuf):
                pltpu.make_async_copy(
                    ir_hbm.at[pl.ds(base + b * _CHUNK, _CHUNK)], ir_v.at[b],
                    sem_i).wait()
                pltpu.make_async_copy(
                    ic_hbm.at[pl.ds(base + b * _CHUNK, _CHUNK)], ic_v.at[b],
                    sem_i).wait()
                pltpu.async_copy(table_hbm.at[ir_v.at[b]], rr_v.at[b], sem_g)
                pltpu.async_copy(table_hbm.at[ic_v.at[b]], rc_v.at[b], sem_g)
            for b in range(nbuf):
                sl = pl.ds(base + b * _CHUNK, _CHUNK)
                pltpu.make_async_copy(
                    table_hbm.at[ir_v.at[b]], rr_v.at[b], sem_g).wait()
                pltpu.make_async_copy(
                    table_hbm.at[ic_v.at[b]], rc_v.at[b], sem_g).wait()
                pltpu.async_copy(rr_v.at[b], gr_hbm.at[sl], sem_o)
                pltpu.async_copy(rc_v.at[b], gc_hbm.at[sl], sem_o)
            for b in range(nbuf):
                sl = pl.ds(base + b * _CHUNK, _CHUNK)
                pltpu.make_async_copy(rr_v.at[b], gr_hbm.at[sl], sem_o).wait()
                pltpu.make_async_copy(rc_v.at[b], gc_hbm.at[sl], sem_o).wait()
            return carry

        lax.fori_loop(0, n_outer, outer, 0)

    return pl.kernel(
        body,
        mesh=mesh,
        out_type=(jax.ShapeDtypeStruct((M, 128), jnp.float32),
                  jax.ShapeDtypeStruct((M, 128), jnp.float32)),
        scratch_types=[
            pltpu.VMEM((nbuf, _CHUNK), jnp.int32),
            pltpu.VMEM((nbuf, _CHUNK), jnp.int32),
            pltpu.VMEM((nbuf, _CHUNK, 128), jnp.float32),
            pltpu.VMEM((nbuf, _CHUNK, 128), jnp.float32),
            pltpu.SemaphoreType.DMA,
            pltpu.SemaphoreType.DMA,
            pltpu.SemaphoreType.DMA,
        ],
    )(table, idx_r, idx_c)


def _relu(v):
    return jnp.maximum(v, 0.0)


def _dot(x, w):
    return lax.dot_general(x, w, (((1,), (0,)), ((), ())),
                           preferred_element_type=jnp.float32)


def _log_softmax(t):
    m = jnp.max(t, axis=1, keepdims=True)
    return t - (m + jnp.log(jnp.sum(jnp.exp(t - m), axis=1, keepdims=True)))


_LKEYS = ("w128", "wc", "be1", "we2", "be2", "wne", "bn1", "wn2", "bn2")
_CKEYS = ("cw1", "cb1", "cw2", "cb2", "ew1", "eb1", "ew2", "eb2")


def _meta_layer(Gcat, ea, x, W, eh):
    m1 = _dot(Gcat, W["w128"])
    he = _relu(m1[:, :eh] + _dot(ea, W["wc"]) + W["be1"])
    ea_new = _dot(he, W["we2"]) + W["be2"]
    if x is not None:
        ea_new = ea_new + ea
    hn = _relu(m1[:, eh:] + _dot(ea_new, W["wne"]) + W["bn1"])
    x_new = _dot(hn, W["wn2"]) + W["bn2"]
    if x is not None:
        x_new = x_new + x
    return ea_new, x_new


def _wrefs(keys, refs):
    return {k: r[...] for k, r in zip(keys, refs)}


def _full_specs(arrs):
    specs = []
    for a in arrs:
        nd = a.ndim
        specs.append(pl.BlockSpec(a.shape, lambda i, _nd=nd: (0,) * _nd))
    return specs


def _prefix_layer(Gr, Gc, ea, xprev, w, eh):
    wlist = [w[k] for k in _LKEYS]
    residual = xprev is not None

    def body(gr_ref, gc_ref, ea_ref, *rest):
        if residual:
            xp_ref, rest = rest[0], rest[1:]
        wr = rest[:len(_LKEYS)]
        ea_o, x_o = rest[len(_LKEYS)], rest[len(_LKEYS) + 1]
        W = _wrefs(_LKEYS, wr)
        x = xp_ref[...] if residual else None
        gcat = jnp.concatenate([gr_ref[...][:, :64], gc_ref[...][:, :64]],
                               axis=1)
        ea_new, x_new = _meta_layer(gcat, ea_ref[...], x, W, eh)
        ea_o[...] = ea_new
        x_o[...] = x_new

    ins = [Gr, Gc, ea] + ([xprev] if residual else []) + wlist
    in_specs = [
        pl.BlockSpec((_B, 128), lambda i: (i, 0)),
        pl.BlockSpec((_B, 128), lambda i: (i, 0)),
        pl.BlockSpec((_B, ea.shape[1]), lambda i: (i, 0)),
    ]
    if residual:
        in_specs.append(pl.BlockSpec((_B, 16), lambda i: (i, 0)))
    in_specs += _full_specs(wlist)
    return pl.pallas_call(
        body,
        grid=(_NP // _B,),
        in_specs=in_specs,
        out_specs=(pl.BlockSpec((_B, 16), lambda i: (i, 0)),
                   pl.BlockSpec((_B, 16), lambda i: (i, 0))),
        out_shape=(jax.ShapeDtypeStruct((_NP, 16), jnp.float32),
                   jax.ShapeDtypeStruct((_NP, 16), jnp.float32)),
    )(*ins)


def _mega(Gr, Gc, ea0, layers, cls, blk0, nblk, ne):
    wlist = [l[k] for l in layers for k in _LKEYS] + [cls[k] for k in _CKEYS]
    nl = len(_LKEYS)

    def body(gr_ref, gc_ref, ea_ref, *rest):
        Ws = [_wrefs(_LKEYS, rest[i * nl:(i + 1) * nl]) for i in range(4)]
        C = _wrefs(_CKEYS, rest[4 * nl:4 * nl + len(_CKEYS)])
        no_ref, eo_ref = rest[-2], rest[-1]
        gcat = jnp.concatenate([gr_ref[...][:, :64], gc_ref[...][:, :64]],
                               axis=1)
        ea, x = ea_ref[...], None
        for i in range(4):
            ea, x = _meta_layer(gcat, ea, x, Ws[i], 32 if i == 0 else 64)
        hn = _relu(_dot(x, C["cw1"]) + C["cb1"])
        tn = _relu(_dot(hn, C["cw2"]) + C["cb2"])
        he = _relu(_dot(ea, C["ew1"]) + C["eb1"])
        te = _relu(_dot(he, C["ew2"]) + C["eb2"])
        no_ref[...] = _log_softmax(tn)
        eo_ref[...] = _log_softmax(te)

    in_specs = [
        pl.BlockSpec((_BM, 128), lambda i: (i, 0)),
        pl.BlockSpec((_BM, 128), lambda i: (i, 0)),
        pl.BlockSpec((_BM, 6), lambda i: (i + blk0, 0)),
    ] + _full_specs(wlist)
    return pl.pallas_call(
        body,
        grid=(nblk,),
        in_specs=in_specs,
        out_specs=(pl.BlockSpec((_BM, 2), lambda i: (i, 0)),
                   pl.BlockSpec((_BM, 4), lambda i: (i, 0))),
        out_shape=(jax.ShapeDtypeStruct((ne, 2), jnp.float32),
                   jax.ShapeDtypeStruct((ne, 4), jnp.float32)),
    )(Gr, Gc, ea0, *wlist)


def _prep_weights(p):
    layers = []
    for i in (1, 2, 3, 4):
        eT = p["e%dW1" % i].T
        nT = p["n%dW1" % i].T
        d = _XDIM[i - 1]
        lo = _XOFF[i - 1]
        eh = eT.shape[1]
        nh = nT.shape[1]
        w128 = jnp.zeros((128, eh + nh), jnp.float32)
        w128 = w128.at[lo:lo + d, :eh].set(eT[0:d])
        w128 = w128.at[64 + lo:64 + lo + d, :eh].set(eT[d:2 * d])
        w128 = w128.at[64 + lo:64 + lo + d, eh:].set(nT[0:d])
        layers.append(dict(
            w128=w128, wc=eT[2 * d:], be1=p["e%db1" % i][None],
            we2=p["e%dW2" % i].T, be2=p["e%db2" % i][None],
            wne=nT[d:], bn1=p["n%db1" % i][None],
            wn2=p["n%dW2" % i].T, bn2=p["n%db2" % i][None],
        ))
    cls = dict(
        cw1=p["cnW1"].T, cb1=p["cnb1"][None],
        cw2=p["cnW2"].T, cb2=p["cnb2"][None],
        ew1=p["ceW1"].T, eb1=p["ceb1"][None],
        ew2=p["ceW2"].T, eb2=p["ceb2"][None],
    )
    return layers, cls


def kernel(x, edge_attr, edge_index, params):
    layers, cls = _prep_weights(params)
    row = edge_index[0]
    col = edge_index[1]

    row_f = jnp.pad(row, (0, _EPAD - _E))
    col_f = jnp.pad(col, (0, _EPAD - _E))
    row_p = jnp.pad(row[:_N], (0, _NP - _N))
    col_p = jnp.pad(col[:_N], (0, _NP - _N))

    T = jnp.zeros((_N, 128), jnp.float32).at[:, 0:4].set(x)
    ea_p = jnp.pad(edge_attr[:_N], ((0, _NP - _N), (0, 0)))
    xp = None
    for i in range(3):
        Grp, Gcp = _sc_gather2(T, row_p, col_p, nbuf=5)
        ea_p, xp = _prefix_layer(Grp, Gcp, ea_p, xp, layers[i],
                                 32 if i == 0 else 64)
        o = _XOFF[i + 1]
        T = T.at[:, o:o + 16].set(xp[:_N])

    e1 = _EPAD // 2
    Gr1, Gc1 = _sc_gather2(T, row_f[:e1], col_f[:e1], nbuf=7)
    Gr2, Gc2 = _sc_gather2(T, row_f[e1:], col_f[e1:], nbuf=7)
    n1, e1o = _mega(Gr1, Gc1, edge_attr, layers, cls, 0, 98, e1)
    n2, e2o = _mega(Gr2, Gc2, edge_attr, layers, cls, 98, 98, _E - e1)
    return (jnp.concatenate([n1, n2]), jnp.concatenate([e1o, e2o]))
